# baseline (speedup 1.0000x reference)
import functools

import jax
import jax.numpy as jnp
from jax import lax
from jax.experimental import pallas as pl
from jax.experimental.pallas import tpu as pltpu
from jax.experimental.pallas import tpu_sc as plsc

N, F, H, B = 100000, 128, 64, 512
BN = 1024
NP = 102400
GRID = NP // BN
LAST = (N + BN - 1) // BN - 1
PV = BN // 4
LASTV = (N + PV - 1) // PV - 1
NW = 32
CPT = NP // NW
WPT = CPT // 128
NCH = 5
CHN = CPT // NCH


def _dense_body(mc_ref, sc_ref, v0_ref, v1_ref, v2_ref, v3_ref, W1_ref,
                b1_ref, W2_ref, b2_ref, wgc_ref, bg_ref, mu_ref):
    step = pl.program_id(0)
    s = sc_ref[...]
    h = jnp.dot(s, W1_ref[...], preferred_element_type=jnp.float32)
    h = h + b1_ref[...]
    h = h * jax.nn.sigmoid(h)
    qg = jnp.dot(h, W2_ref[...], preferred_element_type=jnp.float32)
    qg = qg + b2_ref[...]
    q = qg[:, 0:1]
    gate = qg[:, 1:2]

    wgc = wgc_ref[...]
    parts = [r[...] for r in (v0_ref, v1_ref, v2_ref, v3_ref)]
    vp = [jnp.concatenate(
              [jnp.dot(p[:, c, :], wgc[:, c:c + 1],
                       preferred_element_type=jnp.float32) for p in parts],
              axis=0)
          for c in range(3)]
    vproj = jnp.concatenate(vp, axis=1) + bg_ref[0, 0]

    mu = gate * vproj + q * mc_ref[...]
    row = step * BN + lax.broadcasted_iota(jnp.int32, (BN, 1), 0)
    mu = jnp.where(row < N, mu, 0.0)
    mu_ref[...] = jnp.concatenate(
        [mu, jnp.zeros((BN, 125), jnp.float32)], axis=1)


def _dense_mu(mass_center_vec, scaler, vector, W1, b1, W2, b2, wgc, bg2):
    cl = lambda i: (jnp.minimum(i, LAST), 0)
    P = PV
    vmap4 = [lambda i, k=k: (jnp.minimum(4 * i + k, LASTV), 0, 0)
             for k in range(4)]
    return pl.pallas_call(
        _dense_body,
        grid=(GRID,),
        in_specs=[
            pl.BlockSpec((BN, 3), cl),
            pl.BlockSpec((BN, F), cl),
            pl.BlockSpec((P, 3, F), vmap4[0]),
            pl.BlockSpec((P, 3, F), vmap4[1]),
            pl.BlockSpec((P, 3, F), vmap4[2]),
            pl.BlockSpec((P, 3, F), vmap4[3]),
            pl.BlockSpec((F, 2 * H), lambda i: (0, 0)),
            pl.BlockSpec((1, 2 * H), lambda i: (0, 0)),
            pl.BlockSpec((2 * H, 2), lambda i: (0, 0)),
            pl.BlockSpec((1, 2), lambda i: (0, 0)),
            pl.BlockSpec((F, 3), lambda i: (0, 0)),
            pl.BlockSpec((1, 1), lambda i: (0, 0)),
        ],
        out_specs=pl.BlockSpec((BN, 128), lambda i: (i, 0)),
        out_shape=jax.ShapeDtypeStruct((NP, 128), jnp.float32),
        compiler_params=pltpu.CompilerParams(
            dimension_semantics=("arbitrary",),
        ),
    )(mass_center_vec, scaler, vector, vector, vector, vector,
      W1, b1, W2, b2, wgc, bg2)


def _make_sc_scatter():
    mesh = plsc.VectorSubcoreMesh(core_axis_name="c", subcore_axis_name="s")

    @functools.partial(
        pl.kernel, mesh=mesh,
        out_type=jax.ShapeDtypeStruct((2, B, 128), jnp.float32),
        compiler_params=pltpu.CompilerParams(use_tc_tiling_on_sc=False),
        scratch_types=[
            pltpu.VMEM((WPT, 128), jnp.int32),
            pltpu.VMEM((CHN, 128), jnp.float32),
            pltpu.VMEM_SHARED((B, 128), jnp.float32),
        ],
    )
    def sc_scatter(mu_hbm, idx_hbm, zeros_hbm, out_hbm, idx_v, mu_v, acc_sh):
        c = lax.axis_index("c")
        s = lax.axis_index("s")
        wid = s * 2 + c

        @pl.when(s == 0)
        def _zero():
            pltpu.sync_copy(zeros_hbm, acc_sh)

        pltpu.sync_copy(idx_hbm.at[wid], idx_v)
        plsc.subcore_barrier()
        for ch in range(NCH):
            pltpu.sync_copy(mu_hbm.at[wid, pl.ds(ch * CHN, CHN)], mu_v)
            for jj in range(CHN // 128):
                j = ch * (CHN // 128) + jj
                pltpu.sync_copy(mu_v.at[pl.ds(jj * 128, 128)],
                                acc_sh.at[idx_v.at[j]], add=True)
        plsc.subcore_barrier()

        @pl.when(s == 0)
        def _writeback():
            pltpu.sync_copy(acc_sh, out_hbm.at[c])

    return sc_scatter


def _norm_body(p_ref, out_ref):
    gm = p_ref[0] + p_ref[1]
    out_ref[...] = jnp.sqrt(jnp.sum(gm * gm, axis=1, keepdims=True))


@jax.jit
def kernel(mass_center_vec, scaler, vector, batch_index,
           Wq1, bq1, Wq2, bq2, Wm1, bm1, Wm2, bm2, Wg, bg):
    f32 = jnp.float32
    W1 = jnp.concatenate([Wq1, Wm1], axis=1)
    b1 = jnp.concatenate([bq1, bm1]).reshape(1, 2 * H)
    zH = jnp.zeros((H, 1), f32)
    W2 = jnp.concatenate([jnp.concatenate([Wq2, zH], axis=1),
                          jnp.concatenate([zH, Wm2], axis=1)], axis=0)
    b2 = jnp.concatenate([bq2, bm2]).reshape(1, 2)
    wgc = jnp.tile(Wg, (1, 3))
    bg2 = bg.reshape(1, 1)

    mu = _dense_mu(mass_center_vec, scaler, vector, W1, b1, W2, b2, wgc, bg2)
    mu3 = mu.reshape(NW, CPT, 128)
    idx3 = jnp.pad(batch_index, (0, NP - N)).reshape(NW, WPT, 128)
    zeros = jnp.zeros((B, 128), f32)

    part = _make_sc_scatter()(mu3, idx3, zeros)

    out = pl.pallas_call(
        _norm_body,
        out_shape=jax.ShapeDtypeStruct((B, 1), f32),
    )(part)
    return out


# speedup vs baseline: 1.1114x; 1.1114x over previous
import functools

import jax
import jax.numpy as jnp
from jax import lax
from jax.experimental import pallas as pl
from jax.experimental.pallas import tpu as pltpu

N, F, H, B = 100000, 128, 64, 512
BN = 4000
GRID = N // BN


def _fused_body(mc_ref, sc_ref, v0_ref, v1_ref, v2_ref, v3_ref,
                v4_ref, v5_ref, v6_ref, v7_ref, idx_ref,
                W1_ref, b1_ref, W2_ref, b2_ref, wgc_ref, bg_ref,
                out_ref, acc_ref):
    step = pl.program_id(0)

    @pl.when(step == 0)
    def _init():
        acc_ref[...] = jnp.zeros_like(acc_ref)

    s = sc_ref[...]
    h = jnp.dot(s, W1_ref[...], preferred_element_type=jnp.float32)
    h = h + b1_ref[...]
    h = h * jax.nn.sigmoid(h)
    qg = jnp.dot(h, W2_ref[...], preferred_element_type=jnp.float32)
    qg = qg + b2_ref[...]
    q = qg[:, 0:1]
    gate = qg[:, 1:2]

    wgc = wgc_ref[...]
    refs = (v0_ref, v1_ref, v2_ref, v3_ref,
            v4_ref, v5_ref, v6_ref, v7_ref)
    vp = [jnp.concatenate(
              [jnp.dot(r[:, c, :], wgc[:, c:c + 1],
                       preferred_element_type=jnp.float32) for r in refs],
              axis=0)
          for c in range(3)]
    vproj = jnp.concatenate(vp, axis=1) + bg_ref[0, 0]

    mu = gate * vproj + q * mc_ref[...]

    idx = idx_ref[0, 0, :]
    onehot = (idx[:, None] == lax.broadcasted_iota(
        jnp.int32, (BN, B), 1)).astype(jnp.bfloat16)
    part = lax.dot_general(onehot, mu.astype(jnp.bfloat16),
                           (((0,), (0,)), ((), ())),
                           preferred_element_type=jnp.float32)
    acc_ref[...] += part

    @pl.when(step == GRID - 1)
    def _fin():
        gm = acc_ref[...]
        out_ref[...] = jnp.sqrt(jnp.sum(gm * gm, axis=1, keepdims=True))


@functools.partial(jax.jit, static_argnames=("interpret",))
def kernel(mass_center_vec, scaler, vector, batch_index,
           Wq1, bq1, Wq2, bq2, Wm1, bm1, Wm2, bm2, Wg, bg,
           interpret=False):
    f32 = jnp.float32
    W1 = jnp.concatenate([Wq1, Wm1], axis=1)
    b1 = jnp.concatenate([bq1, bm1]).reshape(1, 2 * H)
    zH = jnp.zeros((H, 1), f32)
    W2 = jnp.concatenate([jnp.concatenate([Wq2, zH], axis=1),
                          jnp.concatenate([zH, Wm2], axis=1)], axis=0)
    b2 = jnp.concatenate([bq2, bm2]).reshape(1, 2)
    wgc = jnp.tile(Wg, (1, 3))
    bg2 = bg.reshape(1, 1)
    idx3 = batch_index.reshape(GRID, 1, BN)

    out = pl.pallas_call(
        _fused_body,
        grid=(GRID,),
        in_specs=[
            pl.BlockSpec((BN, 3), lambda i: (i, 0)),
            pl.BlockSpec((BN, F), lambda i: (i, 0)),
            *[pl.BlockSpec((BN // 8, 3, F), (lambda i, k=k: (8 * i + k, 0, 0)))
              for k in range(8)],
            pl.BlockSpec((1, 1, BN), lambda i: (i, 0, 0)),
            pl.BlockSpec((F, 2 * H), lambda i: (0, 0)),
            pl.BlockSpec((1, 2 * H), lambda i: (0, 0)),
            pl.BlockSpec((2 * H, 2), lambda i: (0, 0)),
            pl.BlockSpec((1, 2), lambda i: (0, 0)),
            pl.BlockSpec((F, 3), lambda i: (0, 0)),
            pl.BlockSpec((1, 1), lambda i: (0, 0)),
        ],
        out_specs=pl.BlockSpec((B, 1), lambda i: (0, 0)),
        out_shape=jax.ShapeDtypeStruct((B, 1), f32),
        scratch_shapes=[pltpu.VMEM((B, 3), f32)],
        compiler_params=pltpu.CompilerParams(
            dimension_semantics=("arbitrary",),
        ),
        interpret=interpret,
    )(mass_center_vec, scaler, *([vector] * 8), idx3,
      W1, b1, W2, b2, wgc, bg2)
    return out


# speedup vs baseline: 1.2705x; 1.1431x over previous
import functools

import jax
import jax.numpy as jnp
from jax import lax
from jax.experimental import pallas as pl
from jax.experimental.pallas import tpu as pltpu

N, F, H, B = 100000, 128, 64, 512
BN = 4000
GRID = N // BN


def _fused_body(mc_ref, sc_ref, v0_ref, v1_ref, v2_ref, v3_ref,
                v4_ref, v5_ref, v6_ref, v7_ref, idx_ref,
                W1_ref, b1_ref, W2_ref, b2_ref, wgc_ref, bg_ref,
                out_ref, acc_ref):
    step = pl.program_id(0)

    @pl.when(step == 0)
    def _init():
        acc_ref[...] = jnp.zeros_like(acc_ref)

    s = sc_ref[...]
    h = jnp.dot(s, W1_ref[...], preferred_element_type=jnp.float32)
    h = h + b1_ref[...]
    h = h * jax.nn.sigmoid(h)
    qg = jnp.dot(h, W2_ref[...], preferred_element_type=jnp.float32)
    qg = qg + b2_ref[...]
    q = qg[:, 0:1]
    gate = qg[:, 1:2]

    wgc = wgc_ref[...]
    refs = (v0_ref, v1_ref, v2_ref, v3_ref,
            v4_ref, v5_ref, v6_ref, v7_ref)
    vproj = jnp.concatenate(
        [jnp.dot(jnp.reshape(r[...], (BN // 8, 3 * F)), wgc,
                 preferred_element_type=jnp.float32) for r in refs],
        axis=0) + bg_ref[0, 0]

    mu = gate * vproj + q * mc_ref[...]

    idx = idx_ref[0, 0, :]
    onehot = (idx[:, None] == lax.broadcasted_iota(
        jnp.int32, (BN, B), 1)).astype(jnp.bfloat16)
    part = lax.dot_general(onehot, mu.astype(jnp.bfloat16),
                           (((0,), (0,)), ((), ())),
                           preferred_element_type=jnp.float32)
    acc_ref[...] += part

    @pl.when(step == GRID - 1)
    def _fin():
        gm = acc_ref[...]
        out_ref[...] = jnp.sqrt(jnp.sum(gm * gm, axis=1, keepdims=True))


@functools.partial(jax.jit, static_argnames=("interpret",))
def kernel(mass_center_vec, scaler, vector, batch_index,
           Wq1, bq1, Wq2, bq2, Wm1, bm1, Wm2, bm2, Wg, bg,
           interpret=False):
    f32 = jnp.float32
    W1 = jnp.concatenate([Wq1, Wm1], axis=1)
    b1 = jnp.concatenate([bq1, bm1]).reshape(1, 2 * H)
    zH = jnp.zeros((H, 1), f32)
    W2 = jnp.concatenate([jnp.concatenate([Wq2, zH], axis=1),
                          jnp.concatenate([zH, Wm2], axis=1)], axis=0)
    b2 = jnp.concatenate([bq2, bm2]).reshape(1, 2)
    zF = jnp.zeros((F, 1), f32)
    wgc = jnp.concatenate(
        [jnp.concatenate([Wg, zF, zF], axis=1),
         jnp.concatenate([zF, Wg, zF], axis=1),
         jnp.concatenate([zF, zF, Wg], axis=1)], axis=0)
    bg2 = bg.reshape(1, 1)
    idx3 = batch_index.reshape(GRID, 1, BN)

    out = pl.pallas_call(
        _fused_body,
        grid=(GRID,),
        in_specs=[
            pl.BlockSpec((BN, 3), lambda i: (i, 0)),
            pl.BlockSpec((BN, F), lambda i: (i, 0)),
            *[pl.BlockSpec((BN // 8, 3, F), (lambda i, k=k: (8 * i + k, 0, 0)))
              for k in range(8)],
            pl.BlockSpec((1, 1, BN), lambda i: (i, 0, 0)),
            pl.BlockSpec((F, 2 * H), lambda i: (0, 0)),
            pl.BlockSpec((1, 2 * H), lambda i: (0, 0)),
            pl.BlockSpec((2 * H, 2), lambda i: (0, 0)),
            pl.BlockSpec((1, 2), lambda i: (0, 0)),
            pl.BlockSpec((3 * F, 3), lambda i: (0, 0)),
            pl.BlockSpec((1, 1), lambda i: (0, 0)),
        ],
        out_specs=pl.BlockSpec((B, 1), lambda i: (0, 0)),
        out_shape=jax.ShapeDtypeStruct((B, 1), f32),
        scratch_shapes=[pltpu.VMEM((B, 3), f32)],
        compiler_params=pltpu.CompilerParams(
            dimension_semantics=("arbitrary",),
        ),
        interpret=interpret,
    )(mass_center_vec, scaler, *([vector] * 8), idx3,
      W1, b1, W2, b2, wgc, bg2)
    return out
